# Initial kernel scaffold; baseline (speedup 1.0000x reference)
import jax
import jax.numpy as jnp
from jax.experimental import pallas as pl


def kernel(x, W, b):
    raise NotImplementedError("write your pallas kernel here")


# speedup vs baseline: 1.0085x; 1.0085x over previous
import functools

import jax
import jax.numpy as jnp
from jax.experimental import pallas as pl


def _router_kernel(x_ref, w_ref, b_ref, probs_ref, weights_ref, *, k, nb, blk, h):
    i = pl.program_id(0)

    xb = x_ref[...].astype(jnp.bfloat16)
    wb = w_ref[...].astype(jnp.bfloat16)
    logits = jnp.dot(xb, wb, preferred_element_type=jnp.float32)
    probs = jax.nn.sigmoid(logits + b_ref[0, 0])
    probs_ref[pl.ds(i, 1), :] = probs.reshape(1, blk)

    @pl.when(i == nb - 1)
    def _():
        bits = probs_ref[...].view(jnp.int32)

        def body(_, lohi):
            lo, hi = lohi
            mid = jax.lax.div(lo + hi, 2)
            cnt = jnp.sum((bits <= mid).astype(jnp.int32))
            return jnp.where(cnt >= k, lo, mid + 1), jnp.where(cnt >= k, mid, hi)

        lo = jnp.int32(0)
        hi = jnp.int32(0x3F800000)
        lo, hi = jax.lax.fori_loop(0, 31, body, (lo, hi))
        weights_ref[...] = (bits >= lo).astype(jnp.float32)


def kernel(x, W, b):
    B, S, H = x.shape
    total = B * S
    capacity = int(total * 0.5)
    k = max(1, total - capacity)

    BLK = 512
    NB = total // BLK
    xf = x.reshape(total, H)
    wt = W.reshape(H, 1)
    b2 = b.reshape(1, 1)

    probs, weights = pl.pallas_call(
        functools.partial(_router_kernel, k=k, nb=NB, blk=BLK, h=H),
        grid=(NB,),
        in_specs=[
            pl.BlockSpec((BLK, H), lambda i: (i, 0)),
            pl.BlockSpec((H, 1), lambda i: (0, 0)),
            pl.BlockSpec((1, 1), lambda i: (0, 0)),
        ],
        out_specs=[
            pl.BlockSpec((NB, BLK), lambda i: (0, 0)),
            pl.BlockSpec((NB, BLK), lambda i: (0, 0)),
        ],
        out_shape=[
            jax.ShapeDtypeStruct((NB, BLK), jnp.float32),
            jax.ShapeDtypeStruct((NB, BLK), jnp.float32),
        ],
    )(xf, wt, b2)

    return (weights.reshape(B, S, 1), probs.reshape(B, S, 1))


# speedup vs baseline: 1.0363x; 1.0276x over previous
import functools

import jax
import jax.numpy as jnp
from jax.experimental import pallas as pl


def _router_kernel(x_ref, w_ref, b_ref, probs_ref, weights_ref, *, k, nb, blk, h):
    i = pl.program_id(0)

    xb = x_ref[...].astype(jnp.bfloat16)
    wb = w_ref[...].astype(jnp.bfloat16)
    logits = jnp.dot(xb, wb, preferred_element_type=jnp.float32)
    probs = jax.nn.sigmoid(logits + b_ref[0, 0])
    probs_ref[pl.ds(i, 1), :] = probs.reshape(1, blk)

    @pl.when(i == nb - 1)
    def _():
        bits = probs_ref[...].view(jnp.int32)

        def body(_, lohi):
            lo, hi = lohi
            mid = jax.lax.div(lo + hi, 2)
            cnt = jnp.sum((bits <= mid).astype(jnp.int32))
            return jnp.where(cnt >= k, lo, mid + 1), jnp.where(cnt >= k, mid, hi)

        lo = jnp.int32(0)
        hi = jnp.int32(0x3F800000)
        lo, hi = jax.lax.fori_loop(0, 31, body, (lo, hi))
        weights_ref[...] = (bits >= lo).astype(jnp.float32)


def kernel(x, W, b):
    B, S, H = x.shape
    total = B * S
    capacity = int(total * 0.5)
    k = max(1, total - capacity)

    BLK = 1024
    NB = total // BLK
    xf = x.reshape(total, H)
    wt = W.reshape(H, 1)
    b2 = b.reshape(1, 1)

    probs, weights = pl.pallas_call(
        functools.partial(_router_kernel, k=k, nb=NB, blk=BLK, h=H),
        grid=(NB,),
        in_specs=[
            pl.BlockSpec((BLK, H), lambda i: (i, 0)),
            pl.BlockSpec((H, 1), lambda i: (0, 0)),
            pl.BlockSpec((1, 1), lambda i: (0, 0)),
        ],
        out_specs=[
            pl.BlockSpec((NB, BLK), lambda i: (0, 0)),
            pl.BlockSpec((NB, BLK), lambda i: (0, 0)),
        ],
        out_shape=[
            jax.ShapeDtypeStruct((NB, BLK), jnp.float32),
            jax.ShapeDtypeStruct((NB, BLK), jnp.float32),
        ],
    )(xf, wt, b2)

    return (weights.reshape(B, S, 1), probs.reshape(B, S, 1))


# speedup vs baseline: 1.0367x; 1.0004x over previous
import functools

import jax
import jax.numpy as jnp
from jax.experimental import pallas as pl


def _router_kernel(xa_ref, xb_ref, w_ref, b_ref, probs_ref, weights_ref,
                   *, k, nb, blk, h):
    i = pl.program_id(0)

    wb = w_ref[...].astype(jnp.bfloat16)
    half = blk // 2
    row = []
    for x_ref in (xa_ref, xb_ref):
        xb16 = x_ref[...].astype(jnp.bfloat16)
        logits = jnp.dot(xb16, wb, preferred_element_type=jnp.float32)
        probs = jax.nn.sigmoid(logits + b_ref[0, 0])
        row.append(probs.reshape(1, half))
    probs_ref[pl.ds(i, 1), :] = jnp.concatenate(row, axis=1)

    @pl.when(i == nb - 1)
    def _():
        bits = probs_ref[...].view(jnp.int32)

        def body(_, lohi):
            lo, hi = lohi
            mid = jax.lax.div(lo + hi, 2)
            cnt = jnp.sum((bits <= mid).astype(jnp.int32))
            return jnp.where(cnt >= k, lo, mid + 1), jnp.where(cnt >= k, mid, hi)

        lo = jnp.int32(0)
        hi = jnp.int32(0x3F800000)
        lo, hi = jax.lax.fori_loop(0, 31, body, (lo, hi))
        weights_ref[...] = (bits >= lo).astype(jnp.float32)


def kernel(x, W, b):
    B, S, H = x.shape
    total = B * S
    capacity = int(total * 0.5)
    k = max(1, total - capacity)

    BLK = 1024
    NB = total // BLK
    HALF = BLK // 2
    xf = x.reshape(total, H)
    wt = W.reshape(H, 1)
    b2 = b.reshape(1, 1)

    probs, weights = pl.pallas_call(
        functools.partial(_router_kernel, k=k, nb=NB, blk=BLK, h=H),
        grid=(NB,),
        in_specs=[
            pl.BlockSpec((HALF, H), lambda i: (2 * i, 0)),
            pl.BlockSpec((HALF, H), lambda i: (2 * i + 1, 0)),
            pl.BlockSpec((H, 1), lambda i: (0, 0)),
            pl.BlockSpec((1, 1), lambda i: (0, 0)),
        ],
        out_specs=[
            pl.BlockSpec((NB, BLK), lambda i: (0, 0)),
            pl.BlockSpec((NB, BLK), lambda i: (0, 0)),
        ],
        out_shape=[
            jax.ShapeDtypeStruct((NB, BLK), jnp.float32),
            jax.ShapeDtypeStruct((NB, BLK), jnp.float32),
        ],
    )(xf, xf, wt, b2)

    return (weights.reshape(B, S, 1), probs.reshape(B, S, 1))
